# Initial kernel scaffold; baseline (speedup 1.0000x reference)
import jax
import jax.numpy as jnp
from jax.experimental import pallas as pl


def kernel(x, a, e, graph_batch, hidden_states, padding_mask, c1_lw, c1_lb, c1_rw, c1_rb, c1_ew, c1_att, c1_b, q_w, q_b, k_w, k_b, v_w, v_b, mha_in_w, mha_in_b, mha_out_w, mha_out_b, cd1_w, cd1_b, c2_lw, c2_lb, c2_rw, c2_rb, c2_ew, c2_att, c2_b, lcm_w, lcm_b, gru_wih, gru_whh, gru_bih, gru_bhh, d1_w, d1_b, o_w, o_b, gln1_w, gln1_b, cln1_w, cln1_b, cln2_w, cln2_b, gln2_w, gln2_b, lnf_w, lnf_b):
    raise NotImplementedError("write your pallas kernel here")


# speedup vs baseline: 5.1344x; 5.1344x over previous
import functools
import math

import jax
import jax.numpy as jnp
from jax import lax
from jax.experimental import pallas as pl
from jax.experimental.pallas import tpu as pltpu
from jax.experimental.pallas import tpu_sc as plsc

B = 8
NP = 1250
N = 10000
EP = 40000
E = B * EP
DF = 128
DE = 16
S = 128
HL = 1024
D = 64
NH = 16
HD = 4
LCM = 1024
OUT = 64

f32 = jnp.float32


def _lin2_body(x_ref, w1_ref, b1_ref, w2_ref, b2_ref, o1_ref, o2_ref):
    xv = x_ref[...]
    o1_ref[...] = jnp.dot(xv, w1_ref[...], preferred_element_type=f32) + b1_ref[...]
    o2_ref[...] = jnp.dot(xv, w2_ref[...], preferred_element_type=f32) + b2_ref[...]


def _lin2(x, w1t, b1, w2t, b2, bm):
    m, k = x.shape
    n = w1t.shape[1]
    grid = m // bm
    return pl.pallas_call(
        _lin2_body,
        grid=(grid,),
        in_specs=[
            pl.BlockSpec((bm, k), lambda i: (i, 0)),
            pl.BlockSpec((k, n), lambda i: (0, 0)),
            pl.BlockSpec((1, n), lambda i: (0, 0)),
            pl.BlockSpec((k, n), lambda i: (0, 0)),
            pl.BlockSpec((1, n), lambda i: (0, 0)),
        ],
        out_specs=[
            pl.BlockSpec((bm, n), lambda i: (i, 0)),
            pl.BlockSpec((bm, n), lambda i: (i, 0)),
        ],
        out_shape=[
            jax.ShapeDtypeStruct((m, n), f32),
            jax.ShapeDtypeStruct((m, n), f32),
        ],
    )(x, w1t, b1.reshape(1, n), w2t, b2.reshape(1, n))


_NW = 32
_PERW = E // _NW
_BLK = 80
_NBLK = _PERW // _BLK
_RPT = N // 16
_RPT8 = 632


def _edge_body(xl_hbm, xr_hbm, ee_hbm, src_hbm, dst_hbm, att_hbm,
               wout_hbm, dout_hbm,
               idxs_v, idxd_v, xlr, xrr, eer, ctr, exr, attv, zb, db,
               sem, wsh, dsh):
    c = lax.axis_index("c")
    s = lax.axis_index("s")

    def zbody(i, carry):
        for j in range(4):
            zb[i, pl.ds(j * 16, 16)] = jnp.zeros((16,), f32)
        db[i, :] = jnp.zeros((16,), f32)
        return carry

    lax.fori_loop(0, 8, zbody, 0)
    soff = pl.multiple_of((s * _RPT) // 8 * 8, 8)

    def zcp(i, carry):
        pltpu.sync_copy(zb, wsh.at[pl.ds(soff + i * 8, 8)])
        pltpu.sync_copy(db, dsh.at[pl.ds(soff + i * 8, 8)])
        return carry

    lax.fori_loop(0, _RPT8 // 8, zcp, 0)
    pltpu.sync_copy(att_hbm, attv)
    plsc.subcore_barrier()

    wid = c * 16 + s

    def blk_body(bi, carry):
        base = pl.multiple_of(wid * _PERW + bi * _BLK, 8)
        pltpu.sync_copy(src_hbm.at[pl.ds(base, _BLK)], idxs_v)
        pltpu.sync_copy(dst_hbm.at[pl.ds(base, _BLK)], idxd_v)
        pltpu.async_copy(xl_hbm.at[idxs_v], xlr, sem).wait()
        pltpu.async_copy(xr_hbm.at[idxd_v], xrr, sem).wait()
        pltpu.sync_copy(ee_hbm.at[pl.ds(base, _BLK)], eer)

        def ebody(ei, ecarry):
            ehot = (lax.iota(jnp.int32, 16) == 0).astype(f32)
            xls = []
            acc = jnp.zeros((16,), f32)
            for j in range(4):
                xlv = xlr[ei, pl.ds(j * 16, 16)]
                xls.append(xlv)
                m = xlv + xrr[ei, pl.ds(j * 16, 16)] + eer[ei, pl.ds(j * 16, 16)]
                ma = jnp.where(m > 0, m, m * 0.01)
                acc = acc + ma * attv[pl.ds(j * 16, 16)]
            ex = jnp.exp(jnp.full((16,), jnp.sum(acc), f32))
            for j in range(4):
                ctr[ei, pl.ds(j * 16, 16)] = ex * xls[j]
            exr[ei, :] = ex * ehot
            return ecarry

        lax.fori_loop(0, _BLK, ebody, 0)
        pltpu.sync_copy(ctr, wsh.at[idxd_v], add=True)
        pltpu.sync_copy(exr, dsh.at[idxd_v], add=True)
        return carry

    lax.fori_loop(0, _NBLK, blk_body, 0)
    plsc.subcore_barrier()

    def ocp(i, carry):
        pltpu.sync_copy(wsh.at[pl.ds(soff + i * 8, 8)],
                        wout_hbm.at[pl.ds(c * N + soff + i * 8, 8)])
        pltpu.sync_copy(dsh.at[pl.ds(soff + i * 8, 8)],
                        dout_hbm.at[pl.ds(c * N + soff + i * 8, 8)])
        return carry

    lax.fori_loop(0, _RPT8 // 8, ocp, 0)


def _edge_agg(xl, xr, ee, src, dst, att):
    mesh = plsc.VectorSubcoreMesh(core_axis_name="c", subcore_axis_name="s")
    fn = functools.partial(
        pl.kernel,
        mesh=mesh,
        out_type=[
            jax.ShapeDtypeStruct((2 * N, D), f32),
            jax.ShapeDtypeStruct((2 * N, 16), f32),
        ],
        scratch_types=[
            pltpu.VMEM((_BLK,), jnp.int32),
            pltpu.VMEM((_BLK,), jnp.int32),
            pltpu.VMEM((_BLK, D), f32),
            pltpu.VMEM((_BLK, D), f32),
            pltpu.VMEM((_BLK, D), f32),
            pltpu.VMEM((_BLK, D), f32),
            pltpu.VMEM((_BLK, 16), f32),
            pltpu.VMEM((D,), f32),
            pltpu.VMEM((8, D), f32),
            pltpu.VMEM((8, 16), f32),
            pltpu.SemaphoreType.DMA,
            pltpu.VMEM_SHARED((N, D), f32),
            pltpu.VMEM_SHARED((N, 16), f32),
        ],
        compiler_params=pltpu.CompilerParams(
            use_tc_tiling_on_sc=False, needs_layout_passes=False),
    )(_edge_body)
    return fn(xl, xr, ee, src, dst, att)


def _post_body(wp_ref, dp_ref, b_ref, gw_ref, gb_ref, o_ref):
    w = wp_ref[0, 0] + wp_ref[1, 0]
    dv = dp_ref[0, 0] + dp_ref[1, 0]
    den = jnp.sum(dv, axis=1, keepdims=True)
    x0 = w / (den + 1e-16) + b_ref[...]
    x1 = jnp.where(x0 > 0, x0, 0.01 * x0)
    mu = jnp.mean(x1)
    xc = x1 - mu
    var = jnp.mean(xc * xc)
    y = xc * lax.rsqrt(var + 1e-5) * gw_ref[...] + gb_ref[...]
    o_ref[0] = y


def _post(wparts, dparts, bias, gw, gb):
    wp = wparts.reshape(2, B, NP, D)
    dp = dparts.reshape(2, B, NP, 16)
    return pl.pallas_call(
        _post_body,
        grid=(B,),
        in_specs=[
            pl.BlockSpec((2, 1, NP, D), lambda b: (0, b, 0, 0)),
            pl.BlockSpec((2, 1, NP, 16), lambda b: (0, b, 0, 0)),
            pl.BlockSpec((1, D), lambda b: (0, 0)),
            pl.BlockSpec((1, D), lambda b: (0, 0)),
            pl.BlockSpec((1, D), lambda b: (0, 0)),
        ],
        out_specs=pl.BlockSpec((1, NP, D), lambda b: (b, 0, 0)),
        out_shape=jax.ShapeDtypeStruct((B, NP, D), f32),
    )(wp, dp, bias.reshape(1, D), gw.reshape(1, D), gb.reshape(1, D))


_CH = NP


def _attn_body(xb_ref, hs_ref, hst_ref,
               qwT_ref, qb_ref, kw_ref, kbc_ref, vwT_ref, vb_ref,
               wqT_ref, bq_ref, wk_ref, bkc_ref, wvT_ref, bv_ref,
               moT_ref, mob_ref, cw1_ref, cb1_ref, cdT_ref, cdb_ref,
               cw2_ref, cb2_ref, o_ref):
    xbs = xb_ref[0]
    hss = hs_ref[0]
    hsts = hst_ref[0]
    q = jnp.dot(xbs, qwT_ref[...], preferred_element_type=f32) + qb_ref[...]
    qh = (jnp.dot(q, wqT_ref[...], preferred_element_type=f32) + bq_ref[...]) * (1.0 / math.sqrt(HD))
    kT = jnp.dot(kw_ref[...], hsts, preferred_element_type=f32) + kbc_ref[...]
    khT = jnp.dot(wk_ref[...], kT, preferred_element_type=f32) + bkc_ref[...]
    v = jnp.dot(hss, vwT_ref[...], preferred_element_type=f32) + vb_ref[...]
    vh = jnp.dot(v, wvT_ref[...], preferred_element_type=f32) + bv_ref[...]

    khTt = jnp.concatenate([khT] * NH, axis=1)
    rk = lax.broadcasted_iota(jnp.int32, (D, NH * S), 0)
    ck = lax.broadcasted_iota(jnp.int32, (D, NH * S), 1)
    kexp = jnp.where((rk // HD) == (ck // S), khTt, 0.0)
    sc = jnp.dot(qh, kexp, preferred_element_type=f32)
    se = jnp.exp(sc)
    rb = lax.broadcasted_iota(jnp.int32, (NH * S, NH), 0)
    cb = lax.broadcasted_iota(jnp.int32, (NH * S, NH), 1)
    bmT = ((rb // S) == cb).astype(f32)
    den = jnp.dot(se, bmT, preferred_element_type=f32)
    rb2 = lax.broadcasted_iota(jnp.int32, (NH, NH * S), 0)
    cb2 = lax.broadcasted_iota(jnp.int32, (NH, NH * S), 1)
    bm = (rb2 == (cb2 // S)).astype(f32)
    denw = jnp.dot(den, bm, preferred_element_type=f32)
    p = se / denw

    vht = jnp.concatenate([vh] * NH, axis=0)
    rv = lax.broadcasted_iota(jnp.int32, (NH * S, D), 0)
    cv = lax.broadcasted_iota(jnp.int32, (NH * S, D), 1)
    vexp = jnp.where((rv // S) == (cv // HD), vht, 0.0)
    ao = jnp.dot(p, vexp, preferred_element_type=f32)
    ao = jnp.dot(ao, moT_ref[...], preferred_element_type=f32) + mob_ref[...]

    x1 = ao + xbs
    mu = jnp.mean(x1, axis=1, keepdims=True)
    xc = x1 - mu
    var = jnp.mean(xc * xc, axis=1, keepdims=True)
    y = xc * lax.rsqrt(var + 1e-5) * cw1_ref[...] + cb1_ref[...]
    h = jnp.dot(y, cdT_ref[...], preferred_element_type=f32) + cdb_ref[...]
    h = jnp.where(h > 0, h, 0.01 * h) + y
    mu2 = jnp.mean(h, axis=1, keepdims=True)
    hc = h - mu2
    var2 = jnp.mean(hc * hc, axis=1, keepdims=True)
    o_ref[0] = hc * lax.rsqrt(var2 + 1e-5) * cw2_ref[...] + cb2_ref[...]


def _attn(xb, hs, hst, p):
    row = lambda a: a.reshape(1, -1)
    col = lambda a: a.reshape(-1, 1)
    wq, wk, wv = jnp.split(p["mha_in_w"], 3, axis=0)
    bq, bk, bv = jnp.split(p["mha_in_b"], 3)
    wspec = lambda shp: pl.BlockSpec(shp, lambda b: (0, 0))
    args = [
        (p["q_w"].T, wspec((D, D))), (row(p["q_b"]), wspec((1, D))),
        (p["k_w"], wspec((D, HL))), (col(p["k_b"]), wspec((D, 1))),
        (p["v_w"].T, wspec((HL, D))), (row(p["v_b"]), wspec((1, D))),
        (wq.T, wspec((D, D))), (row(bq), wspec((1, D))),
        (wk, wspec((D, D))), (col(bk), wspec((D, 1))),
        (wv.T, wspec((D, D))), (row(bv), wspec((1, D))),
        (p["mha_out_w"].T, wspec((D, D))), (row(p["mha_out_b"]), wspec((1, D))),
        (row(p["cln1_w"]), wspec((1, D))), (row(p["cln1_b"]), wspec((1, D))),
        (p["cd1_w"].T, wspec((D, D))), (row(p["cd1_b"]), wspec((1, D))),
        (row(p["cln2_w"]), wspec((1, D))), (row(p["cln2_b"]), wspec((1, D))),
    ]
    return pl.pallas_call(
        _attn_body,
        grid=(B,),
        in_specs=[
            pl.BlockSpec((1, _CH, D), lambda b: (b, 0, 0)),
            pl.BlockSpec((1, S, HL), lambda b: (b, 0, 0)),
            pl.BlockSpec((1, HL, S), lambda b: (b, 0, 0)),
        ] + [sp for _, sp in args],
        out_specs=pl.BlockSpec((1, _CH, D), lambda b: (b, 0, 0)),
        out_shape=jax.ShapeDtypeStruct((B, NP, D), f32),
        compiler_params=pltpu.CompilerParams(vmem_limit_bytes=60 * 1024 * 1024),
    )(xb, hs, hst, *[a for a, _ in args])


def _post2_body(wp_ref, dp_ref, b_ref, gw_ref, gb_ref, lcmT_ref, lcmb_ref, o_ref):
    w = wp_ref[0, 0] + wp_ref[1, 0]
    dv = dp_ref[0, 0] + dp_ref[1, 0]
    den = jnp.sum(dv, axis=1, keepdims=True)
    x0 = w / (den + 1e-16) + b_ref[...]
    x1 = jnp.where(x0 > 0, x0, 0.01 * x0)
    mu = jnp.mean(x1)
    xc = x1 - mu
    var = jnp.mean(xc * xc)
    y = xc * lax.rsqrt(var + 1e-5) * gw_ref[...] + gb_ref[...]
    xp = jnp.dot(y, lcmT_ref[...], preferred_element_type=f32) + lcmb_ref[...]
    o_ref[0] = jnp.maximum(xp, 0.0)


def _post2(wparts, dparts, bias, gw, gb, lcmT, lcmb):
    wp = wparts.reshape(2, B, NP, D)
    dp = dparts.reshape(2, B, NP, 16)
    return pl.pallas_call(
        _post2_body,
        grid=(B,),
        in_specs=[
            pl.BlockSpec((2, 1, NP, D), lambda b: (0, b, 0, 0)),
            pl.BlockSpec((2, 1, NP, 16), lambda b: (0, b, 0, 0)),
            pl.BlockSpec((1, D), lambda b: (0, 0)),
            pl.BlockSpec((1, D), lambda b: (0, 0)),
            pl.BlockSpec((1, D), lambda b: (0, 0)),
            pl.BlockSpec((D, LCM), lambda b: (0, 0)),
            pl.BlockSpec((1, LCM), lambda b: (0, 0)),
        ],
        out_specs=pl.BlockSpec((1, NP, LCM), lambda b: (b, 0, 0)),
        out_shape=jax.ShapeDtypeStruct((B, NP, LCM), f32),
    )(wp, dp, bias.reshape(1, D), gw.reshape(1, D), gb.reshape(1, D),
      lcmT, lcmb.reshape(1, LCM))


def _gru_body(tp_ref, wih_ref, bih_ref, whh_ref, bhh_ref, o_ref):
    blk = tp_ref[...]
    pb = blk.shape[0]
    left = blk[:, 0].reshape(pb * B, LCM)
    right = blk[:, 1].reshape(pb * B, LCM)
    wih = wih_ref[...]
    whh = whh_ref[...]
    bih = bih_ref[...]
    bhh = bhh_ref[...]
    gil = jnp.dot(left, wih, preferred_element_type=f32) + bih
    ghl = jnp.dot(left, whh, preferred_element_type=f32) + bhh
    gir = jnp.dot(right, wih, preferred_element_type=f32) + bih
    ghr = jnp.dot(right, whh, preferred_element_type=f32) + bhh

    def gru(gi, gh, h):
        r = jax.nn.sigmoid(gi[:, :LCM] + gh[:, :LCM])
        z = jax.nn.sigmoid(gi[:, LCM:2 * LCM] + gh[:, LCM:2 * LCM])
        nn_ = jnp.tanh(gi[:, 2 * LCM:] + r * gh[:, 2 * LCM:])
        return (1.0 - z) * nn_ + z * h

    comb = 0.5 * (gru(gil, ghr, right) + gru(gir, ghl, left))
    o_ref[...] = comb.reshape(pb, B, LCM)


def _gru_level(tp, wihT, bih, whhT, bhh):
    p = tp.shape[0]
    pb = min(16, p)
    grid = (p + pb - 1) // pb
    return pl.pallas_call(
        _gru_body,
        grid=(grid,),
        in_specs=[
            pl.BlockSpec((pb, 2, B, LCM), lambda i: (i, 0, 0, 0)),
            pl.BlockSpec((LCM, 3 * LCM), lambda i: (0, 0)),
            pl.BlockSpec((1, 3 * LCM), lambda i: (0, 0)),
            pl.BlockSpec((LCM, 3 * LCM), lambda i: (0, 0)),
            pl.BlockSpec((1, 3 * LCM), lambda i: (0, 0)),
        ],
        out_specs=pl.BlockSpec((pb, B, LCM), lambda i: (i, 0, 0)),
        out_shape=jax.ShapeDtypeStruct((p, B, LCM), f32),
        compiler_params=pltpu.CompilerParams(vmem_limit_bytes=60 * 1024 * 1024),
    )(tp, wihT, bih.reshape(1, 3 * LCM), whhT, bhh.reshape(1, 3 * LCM))


def _head_body(p_ref, d1T_ref, d1b_ref, lw_ref, lb_ref, oT_ref, ob_ref, o_ref):
    h = jnp.dot(p_ref[...], d1T_ref[...], preferred_element_type=f32) + d1b_ref[...]
    h = jnp.where(h > 0, h, 0.01 * h)
    mu = jnp.mean(h, axis=1, keepdims=True)
    hc = h - mu
    var = jnp.mean(hc * hc, axis=1, keepdims=True)
    y = hc * lax.rsqrt(var + 1e-5) * lw_ref[...] + lb_ref[...]
    o_ref[...] = jnp.dot(y, oT_ref[...], preferred_element_type=f32) + ob_ref[...]


def _head(pooled, d1w, d1b, lnfw, lnfb, ow, ob):
    return pl.pallas_call(
        _head_body,
        out_shape=jax.ShapeDtypeStruct((B, OUT), f32),
    )(pooled, d1w.T, d1b.reshape(1, -1), lnfw.reshape(1, -1),
      lnfb.reshape(1, -1), ow.T, ob.reshape(1, -1))


def kernel(x, a, e, graph_batch, hidden_states, padding_mask,
           c1_lw, c1_lb, c1_rw, c1_rb, c1_ew, c1_att, c1_b,
           q_w, q_b, k_w, k_b, v_w, v_b,
           mha_in_w, mha_in_b, mha_out_w, mha_out_b,
           cd1_w, cd1_b,
           c2_lw, c2_lb, c2_rw, c2_rb, c2_ew, c2_att, c2_b,
           lcm_w, lcm_b, gru_wih, gru_whh, gru_bih, gru_bhh,
           d1_w, d1_b, o_w, o_b,
           gln1_w, gln1_b, cln1_w, cln1_b, cln2_w, cln2_b,
           gln2_w, gln2_b, lnf_w, lnf_b):
    xf = x.reshape(N, DF)
    ef = e.reshape(E, DE)
    flat = jnp.transpose(a, (0, 2, 1)).reshape(2, -1).astype(jnp.int32)
    src, dst = flat[0], flat[1]
    hs = hidden_states.reshape(B, S, HL)
    hst = jnp.swapaxes(hs, 1, 2)

    xl1, xr1 = _lin2(xf, c1_lw.T, c1_lb, c1_rw.T, c1_rb, 1000)
    ee1, ee2 = _lin2(ef, c1_ew.T, jnp.zeros((D,), f32), c2_ew.T,
                     jnp.zeros((D,), f32), 4000)

    wparts, dparts = _edge_agg(xl1, xr1, ee1, src, dst, c1_att)
    xb = _post(wparts, dparts, c1_b, gln1_w, gln1_b)

    p = dict(q_w=q_w, q_b=q_b, k_w=k_w, k_b=k_b, v_w=v_w, v_b=v_b,
             mha_in_w=mha_in_w, mha_in_b=mha_in_b, mha_out_w=mha_out_w,
             mha_out_b=mha_out_b, cd1_w=cd1_w, cd1_b=cd1_b,
             cln1_w=cln1_w, cln1_b=cln1_b, cln2_w=cln2_w, cln2_b=cln2_b)
    xb = _attn(xb, hs, hst, p)

    xl2, xr2 = _lin2(xb.reshape(N, D), c2_lw.T, c2_lb, c2_rw.T, c2_rb, 1000)
    wparts2, dparts2 = _edge_agg(xl2, xr2, ee2, src, dst, c2_att)
    t = jnp.swapaxes(
        _post2(wparts2, dparts2, c2_b, gln2_w, gln2_b, lcm_w.T, lcm_b), 0, 1)

    wihT = gru_wih.T
    whhT = gru_whh.T
    while t.shape[0] > 1:
        l = t.shape[0]
        rem = t[l - 1:] if l % 2 == 1 else None
        pcount = l // 2
        tp = t[:2 * pcount].reshape(pcount, 2, B, LCM)
        out = _gru_level(tp, wihT, gru_bih, whhT, gru_bhh)
        t = jnp.concatenate([out, rem], axis=0) if rem is not None else out

    return _head(t[0], d1_w, d1_b, lnf_w, lnf_b, o_w, o_b)
